# Initial kernel scaffold; baseline (speedup 1.0000x reference)
import functools

import jax
import jax.numpy as jnp
from jax import lax
from jax.experimental import pallas as pl
from jax.experimental.pallas import tpu as pltpu
import jax.experimental.pallas.tpu_sc as plsc

NC = 2
NS = 16
NW = NC * NS


@functools.cache
def _build(B, V, D, C):
    assert B % NW == 0
    bpw = B // NW
    assert bpw % C == 0
    steps = bpw // C

    mesh = plsc.VectorSubcoreMesh(
        core_axis_name="c", subcore_axis_name="s",
        num_cores=NC, num_subcores=NS)

    @functools.partial(
        pl.kernel,
        out_type=jax.ShapeDtypeStruct((B, D), jnp.float32),
        mesh=mesh,
        scratch_types=[
            pltpu.VMEM((bpw,), jnp.int32),
            pltpu.VMEM((C, D), jnp.float32),
            pltpu.SemaphoreType.DMA,
        ],
    )
    def gather_kernel(idx_hbm, table_hbm, out_hbm, idx_v, rows_v, sem):
        wid = lax.axis_index("s") * NC + lax.axis_index("c")
        base = wid * bpw
        pltpu.sync_copy(idx_hbm.at[pl.ds(base, bpw)], idx_v)

        @pl.loop(0, steps)
        def _(i):
            off = pl.multiple_of(i * C, C)
            pltpu.async_copy(
                table_hbm.at[idx_v.at[pl.ds(off, C)]], rows_v, sem).wait()
            pltpu.sync_copy(rows_v, out_hbm.at[pl.ds(base + off, C)])

    return gather_kernel


def kernel(input, table):
    V, D = table.shape
    idx = input.reshape(-1).astype(jnp.int32)
    B = idx.shape[0]
    out = _build(B, V, D, 1024)(idx, table)
    return out.reshape(input.shape + (D,))


# speedup vs baseline: 1.8582x; 1.8582x over previous
import functools

import jax
import jax.numpy as jnp
from jax import lax
from jax.experimental import pallas as pl
from jax.experimental.pallas import tpu as pltpu
import jax.experimental.pallas.tpu_sc as plsc

NC = 2
NS = 16
NW = NC * NS


@functools.cache
def _build(B, V, D, C):
    assert B % NW == 0
    bpw = B // NW
    assert bpw % C == 0
    steps = bpw // C

    mesh = plsc.VectorSubcoreMesh(
        core_axis_name="c", subcore_axis_name="s",
        num_cores=NC, num_subcores=NS)

    @functools.partial(
        pl.kernel,
        out_type=jax.ShapeDtypeStruct((B, D), jnp.float32),
        mesh=mesh,
        scratch_types=[
            pltpu.VMEM((bpw,), jnp.int32),
            pltpu.VMEM((C, D), jnp.float32),
            pltpu.SemaphoreType.DMA,
        ],
        compiler_params=pltpu.CompilerParams(use_tc_tiling_on_sc=False),
    )
    def gather_kernel(idx_hbm, table_hbm, out_hbm, idx_v, rows_v, sem):
        wid = lax.axis_index("s") * NC + lax.axis_index("c")
        base = wid * bpw
        pltpu.sync_copy(idx_hbm.at[pl.ds(base, bpw)], idx_v)

        @pl.loop(0, steps)
        def _(i):
            off = pl.multiple_of(i * C, C)
            pltpu.async_copy(
                table_hbm.at[idx_v.at[pl.ds(off, C)]], rows_v, sem).wait()
            pltpu.sync_copy(rows_v, out_hbm.at[pl.ds(base + off, C)])

    return gather_kernel


def kernel(input, table):
    V, D = table.shape
    idx = input.reshape(-1).astype(jnp.int32)
    B = idx.shape[0]
    out = _build(B, V, D, 1024)(idx, table)
    return out.reshape(input.shape + (D,))


# speedup vs baseline: 1.8754x; 1.0092x over previous
import functools

import jax
import jax.numpy as jnp
from jax import lax
from jax.experimental import pallas as pl
from jax.experimental.pallas import tpu as pltpu
import jax.experimental.pallas.tpu_sc as plsc

NC = 2
NS = 16
NW = NC * NS


@functools.cache
def _build(B, V, D, C):
    assert B % NW == 0
    bpw = B // NW
    assert bpw % C == 0
    steps = bpw // C
    assert steps >= 4 and steps % 2 == 0
    outer = (steps - 2) // 2

    mesh = plsc.VectorSubcoreMesh(
        core_axis_name="c", subcore_axis_name="s",
        num_cores=NC, num_subcores=NS)

    @functools.partial(
        pl.kernel,
        out_type=jax.ShapeDtypeStruct((B, D), jnp.float32),
        mesh=mesh,
        scratch_types=[
            pltpu.VMEM((bpw,), jnp.int32),
            pltpu.VMEM((2, C, D), jnp.float32),
            pltpu.SemaphoreType.DMA,
            pltpu.SemaphoreType.DMA,
            pltpu.SemaphoreType.DMA,
            pltpu.SemaphoreType.DMA,
        ],
        compiler_params=pltpu.CompilerParams(use_tc_tiling_on_sc=False),
    )
    def gather_kernel(idx_hbm, table_hbm, out_hbm, idx_v, rows_v,
                      gsem0, gsem1, ssem0, ssem1):
        gsem = (gsem0, gsem1)
        ssem = (ssem0, ssem1)
        wid = lax.axis_index("s") * NC + lax.axis_index("c")
        base = wid * bpw
        pltpu.sync_copy(idx_hbm.at[pl.ds(base, bpw)], idx_v)

        def g_copy(slot, i):
            off = pl.multiple_of(i * C, C)
            return pltpu.make_async_copy(
                table_hbm.at[idx_v.at[pl.ds(off, C)]],
                rows_v.at[slot], gsem[slot])

        def s_copy(slot, i):
            off = pl.multiple_of(i * C, C)
            return pltpu.make_async_copy(
                rows_v.at[slot], out_hbm.at[pl.ds(base + off, C)],
                ssem[slot])

        g_copy(0, 0).start()
        g_copy(1, 1).start()
        g_copy(0, 0).wait()
        s_copy(0, 0).start()

        @pl.loop(0, outer)
        def _(g):
            for k in range(2):
                i = 1 + g * 2 + k
                b = (1 + k) % 2
                s_copy(1 - b, i - 1).wait()
                g_copy(1 - b, i + 1).start()
                g_copy(b, i).wait()
                s_copy(b, i).start()

        i = steps - 1
        s_copy(0, i - 1).wait()
        g_copy(1, i).wait()
        s_copy(1, i).start()
        s_copy(1, i).wait()

    return gather_kernel


def kernel(input, table):
    V, D = table.shape
    idx = input.reshape(-1).astype(jnp.int32)
    B = idx.shape[0]
    out = _build(B, V, D, 640)(idx, table)
    return out.reshape(input.shape + (D,))


# speedup vs baseline: 2.3062x; 1.2297x over previous
import functools

import jax
import jax.numpy as jnp
from jax import lax
from jax.experimental import pallas as pl
from jax.experimental.pallas import tpu as pltpu
import jax.experimental.pallas.tpu_sc as plsc

NC = 2
NS = 16
NW = NC * NS


@functools.cache
def _build_transpose(V, D, CB):
    grid = (V + CB - 1) // CB

    def tr(in_ref, out_ref):
        t = in_ref[...].T
        out_ref[...] = jnp.concatenate([t, t], axis=1)

    return pl.pallas_call(
        tr,
        grid=(grid,),
        in_specs=[pl.BlockSpec((D, CB), lambda j: (0, j))],
        out_specs=pl.BlockSpec((CB, 2 * D), lambda j: (j, 0)),
        out_shape=jax.ShapeDtypeStruct((V, 2 * D), jnp.float32),
    )


@functools.cache
def _build_gather(B, V, D, S, SP, C):
    assert B % NW == 0
    bpw = B // NW
    assert bpw % C == 0
    steps = bpw // C
    assert steps >= 4 and steps % 2 == 0
    outer = (steps - 2) // 2
    rows_out = (B // S) * SP

    mesh = plsc.VectorSubcoreMesh(
        core_axis_name="c", subcore_axis_name="s",
        num_cores=NC, num_subcores=NS)

    @functools.partial(
        pl.kernel,
        out_type=jax.ShapeDtypeStruct((rows_out, 2 * D), jnp.float32),
        mesh=mesh,
        scratch_types=[
            pltpu.VMEM((bpw,), jnp.int32),
            pltpu.VMEM((2, C), jnp.int32),
            pltpu.VMEM((2, C, 2 * D), jnp.float32),
            pltpu.SemaphoreType.DMA,
            pltpu.SemaphoreType.DMA,
            pltpu.SemaphoreType.DMA,
            pltpu.SemaphoreType.DMA,
        ],
        compiler_params=pltpu.CompilerParams(use_tc_tiling_on_sc=False),
    )
    def gather_kernel(idx_hbm, dst_hbm, table_hbm, out_hbm,
                      idx_v, dst_v, rows_v, gsem0, gsem1, ssem0, ssem1):
        gsem = (gsem0, gsem1)
        ssem = (ssem0, ssem1)
        wid = lax.axis_index("s") * NC + lax.axis_index("c")
        base = wid * bpw
        pltpu.sync_copy(idx_hbm.at[pl.ds(base, bpw)], idx_v)

        def load_dst(slot, i):
            off = pl.multiple_of(base + i * C, C)
            pltpu.sync_copy(dst_hbm.at[pl.ds(off, C)], dst_v.at[slot])

        def g_copy(slot, i):
            off = pl.multiple_of(i * C, C)
            return pltpu.make_async_copy(
                table_hbm.at[idx_v.at[pl.ds(off, C)]],
                rows_v.at[slot], gsem[slot])

        def s_copy(slot):
            return pltpu.make_async_copy(
                rows_v.at[slot], out_hbm.at[dst_v.at[slot]], ssem[slot])

        load_dst(0, 0)
        load_dst(1, 1)
        g_copy(0, 0).start()
        g_copy(1, 1).start()
        g_copy(0, 0).wait()
        s_copy(0).start()

        @pl.loop(0, outer)
        def _(g):
            for k in range(2):
                i = 1 + g * 2 + k
                b = (1 + k) % 2
                s_copy(1 - b).wait()
                load_dst(1 - b, i + 1)
                g_copy(1 - b, i + 1).start()
                g_copy(b, i).wait()
                s_copy(b).start()

        s_copy(0).wait()
        g_copy(1, steps - 1).wait()
        s_copy(1).start()
        s_copy(1).wait()

    return gather_kernel


def kernel(input, table):
    V, D = table.shape
    NB3, S = input.shape
    SP = 8 * ((S + 7) // 8)
    idx = input.reshape(-1).astype(jnp.int32)
    B = idx.shape[0]
    p = jnp.arange(B, dtype=jnp.int32)
    dst = (p // S) * SP + (p % S)
    table2 = _build_transpose(V, D, 2048)(table.T)
    out1 = _build_gather(B, V, D, S, SP, 320)(idx, dst, table2)
    out3 = out1.reshape(NB3, SP, 2 * D)
    return out3[:, :S, :D]


# speedup vs baseline: 3.4084x; 1.4780x over previous
import functools

import jax
import jax.numpy as jnp
from jax import lax
from jax.experimental import pallas as pl
from jax.experimental.pallas import tpu as pltpu
import jax.experimental.pallas.tpu_sc as plsc

NC = 2
NS = 16
NW = NC * NS


@functools.cache
def _build_transpose(V, D, CB):
    grid = (V + 2 * CB - 1) // (2 * CB)
    NQ = grid * CB

    def tr(in_ref, out_ref):
        x = in_ref[...]
        out_ref[...] = jnp.concatenate(
            [x[:, :CB].T, x[:, CB:].T], axis=1)

    return pl.pallas_call(
        tr,
        grid=(grid,),
        in_specs=[pl.BlockSpec((D, 2 * CB), lambda j: (0, j))],
        out_specs=pl.BlockSpec((CB, 2 * D), lambda j: (j, 0)),
        out_shape=jax.ShapeDtypeStruct((NQ, 2 * D), jnp.float32),
    )


@functools.cache
def _build_gather(B, V, D, S, SP, C):
    assert B % NW == 0
    bpw = B // NW
    assert bpw % C == 0
    steps = bpw // C
    assert steps >= 4 and steps % 2 == 0
    outer = (steps - 2) // 2
    rows_out = (B // S) * SP * 2

    mesh = plsc.VectorSubcoreMesh(
        core_axis_name="c", subcore_axis_name="s",
        num_cores=NC, num_subcores=NS)

    @functools.partial(
        pl.kernel,
        out_type=jax.ShapeDtypeStruct((rows_out, D), jnp.float32),
        mesh=mesh,
        scratch_types=[
            pltpu.VMEM((bpw,), jnp.int32),
            pltpu.VMEM((2, C), jnp.int32),
            pltpu.VMEM((2, C, D), jnp.float32),
            pltpu.SemaphoreType.DMA,
            pltpu.SemaphoreType.DMA,
            pltpu.SemaphoreType.DMA,
            pltpu.SemaphoreType.DMA,
        ],
        compiler_params=pltpu.CompilerParams(use_tc_tiling_on_sc=False),
    )
    def gather_kernel(idx_hbm, dst_hbm, table_hbm, out_hbm,
                      idx_v, dst_v, rows_v, gsem0, gsem1, ssem0, ssem1):
        gsem = (gsem0, gsem1)
        ssem = (ssem0, ssem1)
        wid = lax.axis_index("s") * NC + lax.axis_index("c")
        base = wid * bpw
        pltpu.sync_copy(idx_hbm.at[pl.ds(base, bpw)], idx_v)

        def load_dst(slot, i):
            off = pl.multiple_of(base + i * C, C)
            pltpu.sync_copy(dst_hbm.at[pl.ds(off, C)], dst_v.at[slot])

        def g_copy(slot, i):
            off = pl.multiple_of(i * C, C)
            return pltpu.make_async_copy(
                table_hbm.at[idx_v.at[pl.ds(off, C)]],
                rows_v.at[slot], gsem[slot])

        def s_copy(slot):
            return pltpu.make_async_copy(
                rows_v.at[slot], out_hbm.at[dst_v.at[slot]], ssem[slot])

        load_dst(0, 0)
        load_dst(1, 1)
        g_copy(0, 0).start()
        g_copy(1, 1).start()
        g_copy(0, 0).wait()
        s_copy(0).start()

        @pl.loop(0, outer)
        def _(g):
            for k in range(2):
                i = 1 + g * 2 + k
                b = (1 + k) % 2
                s_copy(1 - b).wait()
                load_dst(1 - b, i + 1)
                g_copy(1 - b, i + 1).start()
                g_copy(b, i).wait()
                s_copy(b).start()

        s_copy(0).wait()
        g_copy(1, steps - 1).wait()
        s_copy(1).start()
        s_copy(1).wait()

    return gather_kernel


def kernel(input, table):
    V, D = table.shape
    NB3, S = input.shape
    SP = 8 * ((S + 7) // 8)
    idx = input.reshape(-1).astype(jnp.int32)
    B = idx.shape[0]
    p = jnp.arange(B, dtype=jnp.int32)
    dst = ((p // S) * SP + (p % S)) * 2
    CB = 2048
    idx = 2 * (CB * (idx // (2 * CB)) + idx % CB) + (idx // CB) % 2
    table2 = _build_transpose(V, D, CB)(table.T)
    V2 = 2 * table2.shape[0]
    table_lin = table2.reshape(-1).reshape(V2, D)
    out1 = _build_gather(B, V2, D, S, SP, 640)(idx, dst, table_lin)
    out3 = out1.reshape(-1).reshape(NB3, SP, 2 * D)
    return out3[:, :S, :D]


# speedup vs baseline: 3.7744x; 1.1074x over previous
import functools

import jax
import jax.numpy as jnp
from jax import lax
from jax.experimental import pallas as pl
from jax.experimental.pallas import tpu as pltpu
import jax.experimental.pallas.tpu_sc as plsc

NC = 2
NS = 16
NW = NC * NS


@functools.cache
def _build_transpose(V, D, CB):
    grid = (V + 2 * CB - 1) // (2 * CB)
    NQ = grid * CB

    def tr(in_ref, out_ref):
        x = in_ref[...]
        out_ref[...] = jnp.concatenate(
            [x[:, :CB].T, x[:, CB:].T], axis=1)

    return pl.pallas_call(
        tr,
        grid=(grid,),
        in_specs=[pl.BlockSpec((D, 2 * CB), lambda j: (0, j))],
        out_specs=pl.BlockSpec((CB, 2 * D), lambda j: (j, 0)),
        out_shape=jax.ShapeDtypeStruct((NQ, 2 * D), jnp.float32),
    )


@functools.cache
def _build_gather(B, V, D, S, SP, C):
    assert B % NW == 0
    bpw = B // NW
    assert bpw % C == 0
    steps = bpw // C
    assert steps >= 4 and steps % 2 == 0
    outer = (steps - 2) // 2
    rows_out = (B // S) * SP * 2

    mesh = plsc.VectorSubcoreMesh(
        core_axis_name="c", subcore_axis_name="s",
        num_cores=NC, num_subcores=NS)

    @functools.partial(
        pl.kernel,
        out_type=jax.ShapeDtypeStruct((rows_out, D), jnp.float32),
        mesh=mesh,
        scratch_types=[
            pltpu.VMEM((bpw,), jnp.int32),
            pltpu.VMEM((2, C), jnp.int32),
            pltpu.VMEM((2, C, D), jnp.float32),
            pltpu.SemaphoreType.DMA,
            pltpu.SemaphoreType.DMA,
            pltpu.SemaphoreType.DMA,
            pltpu.SemaphoreType.DMA,
        ],
        compiler_params=pltpu.CompilerParams(use_tc_tiling_on_sc=False),
    )
    def gather_kernel(idx_hbm, dst_hbm, table_hbm, out_hbm,
                      idx_v, dst_v, rows_v, gsem0, gsem1, ssem0, ssem1):
        gsem = (gsem0, gsem1)
        ssem = (ssem0, ssem1)
        wid = lax.axis_index("s") * NC + lax.axis_index("c")
        base = wid * bpw
        pltpu.sync_copy(idx_hbm.at[pl.ds(base, bpw)], idx_v)

        def load_dst(slot, i):
            off = pl.multiple_of(base + i * C, C)
            pltpu.sync_copy(dst_hbm.at[pl.ds(off, C)], dst_v.at[slot])

        def g_copy(slot, i):
            off = pl.multiple_of(i * C, C)
            return pltpu.make_async_copy(
                table_hbm.at[idx_v.at[pl.ds(off, C)]],
                rows_v.at[slot], gsem[slot])

        def s_copy(slot):
            return pltpu.make_async_copy(
                rows_v.at[slot], out_hbm.at[dst_v.at[slot]], ssem[slot])

        load_dst(0, 0)
        load_dst(1, 1)
        g_copy(0, 0).start()
        g_copy(1, 1).start()
        g_copy(0, 0).wait()
        s_copy(0).start()

        @pl.loop(0, outer)
        def _(g):
            for k in range(2):
                i = 1 + g * 2 + k
                b = (1 + k) % 2
                s_copy(1 - b).wait()
                load_dst(1 - b, i + 1)
                g_copy(1 - b, i + 1).start()
                g_copy(b, i).wait()
                s_copy(b).start()

        s_copy(0).wait()
        g_copy(1, steps - 1).wait()
        s_copy(1).start()
        s_copy(1).wait()

    return gather_kernel


def kernel(input, table):
    V, D = table.shape
    NB3, S = input.shape
    SP = 8 * ((S + 7) // 8)
    idx = input.reshape(-1).astype(jnp.int32)
    B = idx.shape[0]
    p = jnp.arange(B, dtype=jnp.int32)
    dst = ((p // S) * SP + (p % S)) * 2
    CB = 4096
    idx = 2 * (CB * (idx // (2 * CB)) + idx % CB) + (idx // CB) % 2
    table2 = _build_transpose(V, D, CB)(table.T)
    V2 = 2 * table2.shape[0]
    table_lin = table2.reshape(-1).reshape(V2, D)
    out1 = _build_gather(B, V2, D, S, SP, 640)(idx, dst, table_lin)
    out3 = out1.reshape(-1).reshape(NB3, SP, 2 * D)
    return out3[:, :S, :D]


# speedup vs baseline: 3.9732x; 1.0527x over previous
import functools

import jax
import jax.numpy as jnp
from jax import lax
from jax.experimental import pallas as pl
from jax.experimental.pallas import tpu as pltpu
import jax.experimental.pallas.tpu_sc as plsc

NC = 2
NS = 16
NW = NC * NS


@functools.cache
def _build_transpose(V, D, CB):
    grid = (V + 2 * CB - 1) // (2 * CB)
    NQ = grid * CB

    def tr(in_ref, out_ref):
        x = in_ref[...]
        out_ref[...] = jnp.concatenate(
            [x[:, :CB].T, x[:, CB:].T], axis=1)

    return pl.pallas_call(
        tr,
        grid=(grid,),
        in_specs=[pl.BlockSpec((D, 2 * CB), lambda j: (0, j))],
        out_specs=pl.BlockSpec((CB, 2 * D), lambda j: (j, 0)),
        out_shape=jax.ShapeDtypeStruct((NQ, 2 * D), jnp.float32),
    )


@functools.cache
def _build_gather(B, V, D, S, SP, C):
    assert B % NW == 0
    bpw = B // NW
    assert bpw % C == 0
    steps = bpw // C
    assert steps >= 4 and steps % 2 == 0
    outer = (steps - 2) // 2
    rows_out = (B // S) * SP * 2

    mesh = plsc.VectorSubcoreMesh(
        core_axis_name="c", subcore_axis_name="s",
        num_cores=NC, num_subcores=NS)

    @functools.partial(
        pl.kernel,
        out_type=jax.ShapeDtypeStruct((rows_out, D), jnp.float32),
        mesh=mesh,
        scratch_types=[
            pltpu.VMEM((bpw,), jnp.int32),
            pltpu.VMEM((2, C), jnp.int32),
            pltpu.VMEM((2, C, D), jnp.float32),
            pltpu.SemaphoreType.DMA,
            pltpu.SemaphoreType.DMA,
            pltpu.SemaphoreType.DMA,
            pltpu.SemaphoreType.DMA,
        ],
        compiler_params=pltpu.CompilerParams(use_tc_tiling_on_sc=False),
    )
    def gather_kernel(idx_hbm, dst_hbm, table_hbm, out_hbm,
                      idx_v, dst_v, rows_v, gsem0, gsem1, ssem0, ssem1):
        gsem = (gsem0, gsem1)
        ssem = (ssem0, ssem1)
        wid = lax.axis_index("s") * NC + lax.axis_index("c")
        base = wid * bpw
        pltpu.sync_copy(idx_hbm.at[pl.ds(base, bpw)], idx_v)

        def load_dst(slot, i):
            off = pl.multiple_of(base + i * C, C)
            pltpu.sync_copy(dst_hbm.at[pl.ds(off, C)], dst_v.at[slot])

        def g_copy(slot, i):
            off = pl.multiple_of(i * C, C)
            return pltpu.make_async_copy(
                table_hbm.at[idx_v.at[pl.ds(off, C)]],
                rows_v.at[slot], gsem[slot])

        def s_copy(slot):
            return pltpu.make_async_copy(
                rows_v.at[slot], out_hbm.at[dst_v.at[slot]], ssem[slot])

        load_dst(0, 0)
        load_dst(1, 1)
        g_copy(0, 0).start()
        g_copy(1, 1).start()
        g_copy(0, 0).wait()
        s_copy(0).start()

        @pl.loop(0, outer)
        def _(g):
            for k in range(2):
                i = 1 + g * 2 + k
                b = (1 + k) % 2
                s_copy(1 - b).wait()
                load_dst(1 - b, i + 1)
                g_copy(1 - b, i + 1).start()
                g_copy(b, i).wait()
                s_copy(b).start()

        s_copy(0).wait()
        g_copy(1, steps - 1).wait()
        s_copy(1).start()
        s_copy(1).wait()

    return gather_kernel


def kernel(input, table):
    V, D = table.shape
    NB3, S = input.shape
    SP = 8 * ((S + 7) // 8)
    idx = input.reshape(-1).astype(jnp.int32)
    B = idx.shape[0]
    p = jnp.arange(B, dtype=jnp.int32)
    dst = ((p // S) * SP + (p % S)) * 2
    CB = 8192
    idx = 2 * (CB * (idx // (2 * CB)) + idx % CB) + (idx // CB) % 2
    table2 = _build_transpose(V, D, CB)(table.T)
    V2 = 2 * table2.shape[0]
    table_lin = table2.reshape(-1).reshape(V2, D)
    out1 = _build_gather(B, V2, D, S, SP, 640)(idx, dst, table_lin)
    out3 = out1.reshape(-1).reshape(NB3, SP, 2 * D)
    return out3[:, :S, :D]


# speedup vs baseline: 4.0764x; 1.0260x over previous
import functools

import jax
import jax.numpy as jnp
from jax import lax
from jax.experimental import pallas as pl
from jax.experimental.pallas import tpu as pltpu
import jax.experimental.pallas.tpu_sc as plsc

NC = 2
NS = 16
NW = NC * NS


@functools.cache
def _build_transpose(V, D, CB):
    grid = (V + 2 * CB - 1) // (2 * CB)
    NQ = grid * CB

    def tr(in_ref, out_ref):
        x = in_ref[...]
        out_ref[...] = jnp.concatenate(
            [x[:, :CB].T, x[:, CB:].T], axis=1)

    return pl.pallas_call(
        tr,
        grid=(grid,),
        in_specs=[pl.BlockSpec((D, 2 * CB), lambda j: (0, j))],
        out_specs=pl.BlockSpec((CB, 2 * D), lambda j: (j, 0)),
        out_shape=jax.ShapeDtypeStruct((NQ, 2 * D), jnp.float32),
    )


@functools.cache
def _build_gather(B, V, D, S, SP, C):
    assert B % NW == 0
    bpw = B // NW
    assert bpw % C == 0
    steps = bpw // C
    assert steps >= 4 and steps % 2 == 0
    outer = (steps - 2) // 2
    rows_out = (B // S) * SP * 2

    mesh = plsc.VectorSubcoreMesh(
        core_axis_name="c", subcore_axis_name="s",
        num_cores=NC, num_subcores=NS)

    @functools.partial(
        pl.kernel,
        out_type=jax.ShapeDtypeStruct((rows_out, D), jnp.float32),
        mesh=mesh,
        scratch_types=[
            pltpu.VMEM((bpw,), jnp.int32),
            pltpu.VMEM((2, C), jnp.int32),
            pltpu.VMEM((2, C, D), jnp.float32),
            pltpu.SemaphoreType.DMA,
            pltpu.SemaphoreType.DMA,
            pltpu.SemaphoreType.DMA,
            pltpu.SemaphoreType.DMA,
        ],
        compiler_params=pltpu.CompilerParams(use_tc_tiling_on_sc=False),
    )
    def gather_kernel(idx_hbm, dst_hbm, table_hbm, out_hbm,
                      idx_v, dst_v, rows_v, gsem0, gsem1, ssem0, ssem1):
        gsem = (gsem0, gsem1)
        ssem = (ssem0, ssem1)
        wid = lax.axis_index("s") * NC + lax.axis_index("c")
        base = wid * bpw
        pltpu.sync_copy(idx_hbm.at[pl.ds(base, bpw)], idx_v)

        def load_dst(slot, i):
            off = pl.multiple_of(base + i * C, C)
            pltpu.sync_copy(dst_hbm.at[pl.ds(off, C)], dst_v.at[slot])

        def g_copy(slot, i):
            off = pl.multiple_of(i * C, C)
            return pltpu.make_async_copy(
                table_hbm.at[idx_v.at[pl.ds(off, C)]],
                rows_v.at[slot], gsem[slot])

        def s_copy(slot):
            return pltpu.make_async_copy(
                rows_v.at[slot], out_hbm.at[dst_v.at[slot]], ssem[slot])

        load_dst(0, 0)
        load_dst(1, 1)
        g_copy(0, 0).start()
        g_copy(1, 1).start()
        g_copy(0, 0).wait()
        s_copy(0).start()

        @pl.loop(0, outer)
        def _(g):
            for k in range(2):
                i = 1 + g * 2 + k
                b = (1 + k) % 2
                s_copy(1 - b).wait()
                load_dst(1 - b, i + 1)
                g_copy(1 - b, i + 1).start()
                g_copy(b, i).wait()
                s_copy(b).start()

        s_copy(0).wait()
        g_copy(1, steps - 1).wait()
        s_copy(1).start()
        s_copy(1).wait()

    return gather_kernel


def kernel(input, table):
    V, D = table.shape
    NB3, S = input.shape
    SP = 8 * ((S + 7) // 8)
    idx = input.reshape(-1).astype(jnp.int32)
    B = idx.shape[0]
    p = jnp.arange(B, dtype=jnp.int32)
    dst = ((p // S) * SP + (p % S)) * 2
    CB = 16384
    idx = 2 * (CB * (idx // (2 * CB)) + idx % CB) + (idx // CB) % 2
    table2 = _build_transpose(V, D, CB)(table.T)
    V2 = 2 * table2.shape[0]
    table_lin = table2.reshape(-1).reshape(V2, D)
    out1 = _build_gather(B, V2, D, S, SP, 640)(idx, dst, table_lin)
    out3 = out1.reshape(-1).reshape(NB3, SP, 2 * D)
    return out3[:, :S, :D]


# speedup vs baseline: 4.0797x; 1.0008x over previous
import functools

import jax
import jax.numpy as jnp
from jax import lax
from jax.experimental import pallas as pl
from jax.experimental.pallas import tpu as pltpu
import jax.experimental.pallas.tpu_sc as plsc

NC = 2
NS = 16
NW = NC * NS


@functools.cache
def _build_transpose(V, D, CB):
    grid = (V + 2 * CB - 1) // (2 * CB)
    NQ = grid * CB

    def tr(in_ref, out_ref):
        out_ref[:, :D] = in_ref[:, :CB].T
        out_ref[:, D:] = in_ref[:, CB:].T

    return pl.pallas_call(
        tr,
        grid=(grid,),
        in_specs=[pl.BlockSpec((D, 2 * CB), lambda j: (0, j))],
        out_specs=pl.BlockSpec((CB, 2 * D), lambda j: (j, 0)),
        out_shape=jax.ShapeDtypeStruct((NQ, 2 * D), jnp.float32),
    )


@functools.cache
def _build_gather(B, V, D, S, SP, C):
    assert B % NW == 0
    bpw = B // NW
    assert bpw % C == 0
    steps = bpw // C
    assert steps >= 4 and steps % 2 == 0
    outer = (steps - 2) // 2
    rows_out = (B // S) * SP * 2

    mesh = plsc.VectorSubcoreMesh(
        core_axis_name="c", subcore_axis_name="s",
        num_cores=NC, num_subcores=NS)

    @functools.partial(
        pl.kernel,
        out_type=jax.ShapeDtypeStruct((rows_out, D), jnp.float32),
        mesh=mesh,
        scratch_types=[
            pltpu.VMEM((bpw,), jnp.int32),
            pltpu.VMEM((2, C), jnp.int32),
            pltpu.VMEM((2, C, D), jnp.float32),
            pltpu.SemaphoreType.DMA,
            pltpu.SemaphoreType.DMA,
            pltpu.SemaphoreType.DMA,
            pltpu.SemaphoreType.DMA,
        ],
        compiler_params=pltpu.CompilerParams(use_tc_tiling_on_sc=False),
    )
    def gather_kernel(idx_hbm, dst_hbm, table_hbm, out_hbm,
                      idx_v, dst_v, rows_v, gsem0, gsem1, ssem0, ssem1):
        gsem = (gsem0, gsem1)
        ssem = (ssem0, ssem1)
        wid = lax.axis_index("s") * NC + lax.axis_index("c")
        base = wid * bpw
        pltpu.sync_copy(idx_hbm.at[pl.ds(base, bpw)], idx_v)

        def load_dst(slot, i):
            off = pl.multiple_of(base + i * C, C)
            pltpu.sync_copy(dst_hbm.at[pl.ds(off, C)], dst_v.at[slot])

        def g_copy(slot, i):
            off = pl.multiple_of(i * C, C)
            return pltpu.make_async_copy(
                table_hbm.at[idx_v.at[pl.ds(off, C)]],
                rows_v.at[slot], gsem[slot])

        def s_copy(slot):
            return pltpu.make_async_copy(
                rows_v.at[slot], out_hbm.at[dst_v.at[slot]], ssem[slot])

        load_dst(0, 0)
        load_dst(1, 1)
        g_copy(0, 0).start()
        g_copy(1, 1).start()
        g_copy(0, 0).wait()
        s_copy(0).start()

        @pl.loop(0, outer)
        def _(g):
            for k in range(2):
                i = 1 + g * 2 + k
                b = (1 + k) % 2
                s_copy(1 - b).wait()
                load_dst(1 - b, i + 1)
                g_copy(1 - b, i + 1).start()
                g_copy(b, i).wait()
                s_copy(b).start()

        s_copy(0).wait()
        g_copy(1, steps - 1).wait()
        s_copy(1).start()
        s_copy(1).wait()

    return gather_kernel


def kernel(input, table):
    V, D = table.shape
    NB3, S = input.shape
    SP = 8 * ((S + 7) // 8)
    idx = input.reshape(-1).astype(jnp.int32)
    B = idx.shape[0]
    p = jnp.arange(B, dtype=jnp.int32)
    dst = ((p // S) * SP + (p % S)) * 2
    CB = 16384
    idx = 2 * (CB * (idx // (2 * CB)) + idx % CB) + (idx // CB) % 2
    table2 = _build_transpose(V, D, CB)(table.T)
    V2 = 2 * table2.shape[0]
    table_lin = table2.reshape(-1).reshape(V2, D)
    out1 = _build_gather(B, V2, D, S, SP, 640)(idx, dst, table_lin)
    out3 = out1.reshape(-1).reshape(NB3, SP, 2 * D)
    return out3[:, :S, :D]
